# baseline (speedup 1.0000x reference)
import jax
import jax.numpy as jnp
import numpy as np
from jax import lax
from jax.experimental import pallas as pl
from jax.experimental.pallas import tpu as pltpu
from jax.experimental.pallas import tpu_sc as plsc

_N = 10000
_E = 100000
_HD = 128
_H = 4
_D = 32
_C = 153
_NEG = 0.2
_EPS = 1e-9

_NC, _NS, _L = 2, 16, 16
_NW = _NC * _NS
_CHUNK = 96
_NCH = 34
_EPW = _NCH * _CHUNK
_EPAD = _NW * _EPW
_WW = 16
_NPAD = 10112
_RPT = _NPAD // _NS
_RPC = 79

_BLK = 400
_GRID = _N // _BLK


def _transform(hb, w, av, rv, s1, s2, zo, eo):
    z = jnp.dot(hb, w[...], preferred_element_type=jnp.float32)
    zo[...] = z
    eo[...] = (jnp.dot(z * av[...], s1[...], preferred_element_type=jnp.float32)
               + jnp.dot(z * rv[...], s2[...], preferred_element_type=jnp.float32))


def _tc1_body(h, w0, av0, rv0, w1, av1, rv1, w2, av2, rv2, s1, s2,
              z0, e0, z1, e1, z2, e2):
    hb = h[...]
    for w, av, rv, zo, eo in ((w0, av0, rv0, z0, e0), (w1, av1, rv1, z1, e1),
                              (w2, av2, rv2, z2, e2)):
        _transform(hb, w, av, rv, s1, s2, zo, eo)


def _combine(zaccs, waccs, b0, b1, b2, rsel):
    out = b0[...] + b1[...] + b2[...]
    for (z_lo, z_hi), (w_lo, w_hi) in zip(zaccs, waccs):
        nm = z_lo[...] + z_hi[...]
        den = jnp.dot(w_lo[...] + w_hi[...], rsel[...],
                      preferred_element_type=jnp.float32)
        out = out + nm / (den + _EPS)
    return out


def _tc2_body(a00, a01, a10, a11, a20, a21, q00, q01, q10, q11, q20, q21,
              b0, b1, b2, rsel,
              w0, av0, rv0, w1, av1, rv1, w2, av2, rv2, s1, s2,
              z0, e0, z1, e1, z2, e2):
    hb = jnp.maximum(_combine(((a00, a01), (a10, a11), (a20, a21)),
                              ((q00, q01), (q10, q11), (q20, q21)),
                              b0, b1, b2, rsel), 0.0)
    for w, av, rv, zo, eo in ((w0, av0, rv0, z0, e0), (w1, av1, rv1, z1, e1),
                              (w2, av2, rv2, z2, e2)):
        _transform(hb, w, av, rv, s1, s2, zo, eo)


def _tc3_body(a00, a01, a10, a11, a20, a21, q00, q01, q10, q11, q20, q21,
              b0, b1, b2, rsel, lw, lb, out):
    hb = _combine(((a00, a01), (a10, a11), (a20, a21)),
                  ((q00, q01), (q10, q11), (q20, q21)), b0, b1, b2, rsel)
    out[...] = jnp.dot(hb, lw[...], preferred_element_type=jnp.float32) + lb[...]


_FULL = lambda shape: pl.BlockSpec(shape, lambda i: (0,) * len(shape))
_ROWB = lambda shape: pl.BlockSpec(shape, lambda i: (i,) + (0,) * (len(shape) - 1))

_WSPECS = [_FULL((_HD, _HD)), _FULL((1, _HD)), _FULL((1, _HD))] * 3 + \
          [_FULL((_HD, 16)), _FULL((_HD, 16))]
_ZOUTS = [jax.ShapeDtypeStruct((_N, _HD), jnp.float32),
          jax.ShapeDtypeStruct((_N, 16), jnp.float32)] * 3
_ZSPEC = [_ROWB((_BLK, _HD)), _ROWB((_BLK, 16))] * 3
_ACCSPECS = [_ROWB((_BLK, _HD))] * 6 + [_ROWB((_BLK, _WW))] * 6 + \
            [_FULL((1, _HD))] * 3 + [_FULL((16, _HD))]

_tc1 = pl.pallas_call(
    _tc1_body, grid=(_GRID,),
    in_specs=[_ROWB((_BLK, _HD))] + _WSPECS,
    out_specs=_ZSPEC, out_shape=_ZOUTS)

_tc2 = pl.pallas_call(
    _tc2_body, grid=(_GRID,),
    in_specs=_ACCSPECS + _WSPECS,
    out_specs=_ZSPEC, out_shape=_ZOUTS)

_tc3 = pl.pallas_call(
    _tc3_body, grid=(_GRID,),
    in_specs=_ACCSPECS + [_FULL((_HD, 256)), _FULL((1, 256))],
    out_specs=_ROWB((_BLK, 256)),
    out_shape=jax.ShapeDtypeStruct((_N, 256), jnp.float32))


def _sc_body(z_h, eler_h, src_h, dst_h, oz_h, ow_h,
             sidx, didx, zbuf0, esb0, edb0, zbuf1, esb1, edb1,
             wbuf, accz, accw, sem0, sem1):
    c = lax.axis_index("c")
    s = lax.axis_index("s")
    wid = c * _NS + s
    zbuf = (zbuf0, zbuf1)
    esb = (esb0, esb1)
    edb = (edb0, edb1)
    sem = (sem0, sem1)
    pltpu.sync_copy(src_h.at[wid], sidx)
    pltpu.sync_copy(dst_h.at[wid], didx)

    def _zrow(i, carry):
        for j in range(_HD // _L):
            zbuf0[i, pl.ds(j * _L, _L)] = jnp.zeros((_L,), jnp.float32)
        wbuf[i, pl.ds(0, _L)] = jnp.zeros((_L,), jnp.float32)
        return carry
    lax.fori_loop(0, _CHUNK, _zrow, 0)
    for k in range(_RPT // _RPC):
        r0 = s * _RPT + k * _RPC
        pltpu.sync_copy(zbuf0.at[pl.ds(0, _RPC)], accz.at[pl.ds(r0, _RPC)])
        pltpu.sync_copy(wbuf.at[pl.ds(0, _RPC)], accw.at[pl.ds(r0, _RPC)])
    plsc.subcore_barrier()

    def _issue(j, b):
        pltpu.async_copy(z_h.at[sidx.at[j]], zbuf[b], sem[b])
        pltpu.async_copy(eler_h.at[sidx.at[j]], esb[b], sem[b])
        pltpu.async_copy(eler_h.at[didx.at[j]], edb[b], sem[b])

    def _process(j, b):
        pltpu.make_async_copy(z_h.at[sidx.at[j]], zbuf[b], sem[b]).wait()
        pltpu.make_async_copy(eler_h.at[sidx.at[j]], esb[b], sem[b]).wait()
        pltpu.make_async_copy(eler_h.at[didx.at[j]], edb[b], sem[b]).wait()
        gbase = wid * _EPW + j * _CHUNK
        for g in range(_CHUNK // _L):
            rows = lax.iota(jnp.int32, _L) + (g * _L)
            live = (gbase + g * _L + lax.iota(jnp.int32, _L)) < _E
            for h in range(_H):
                a = plsc.load_gather(esb[b], [rows, jnp.full((_L,), h, jnp.int32)])
                bb = plsc.load_gather(edb[b], [rows, jnp.full((_L,), _H + h, jnp.int32)])
                e = a + bb
                e = jnp.maximum(e, _NEG * e)
                w = jnp.where(live, jnp.exp(e), 0.0)
                plsc.store_scatter(wbuf, [rows, jnp.full((_L,), h, jnp.int32)], w)

        def _scale(i, carry2):
            wrow = wbuf[i, pl.ds(0, _L)]
            zb = zbuf[b]
            for h in range(_H):
                wv = wrow[h]
                for tt in range(_D // _L):
                    sl = pl.ds(h * _D + tt * _L, _L)
                    zb[i, sl] = zb[i, sl] * wv
            return carry2
        lax.fori_loop(0, _CHUNK, _scale, 0, unroll=4)

        pltpu.sync_copy(zbuf[b], accz.at[didx.at[j]], add=True)
        pltpu.sync_copy(wbuf, accw.at[didx.at[j]], add=True)

    _issue(0, 0)

    def _pair(jj, carry):
        j0 = jj * 2
        _issue(j0 + 1, 1)
        _process(j0, 0)

        @pl.when(jj < _NCH // 2 - 1)
        def _():
            _issue(j0 + 2, 0)
        _process(j0 + 1, 1)
        return carry
    lax.fori_loop(0, _NCH // 2, _pair, 0)

    plsc.subcore_barrier()
    for k in range(_RPT // _RPC):
        r0 = s * _RPT + k * _RPC
        pltpu.sync_copy(accz.at[pl.ds(r0, _RPC)], zbuf0.at[pl.ds(0, _RPC)])
        pltpu.sync_copy(zbuf0.at[pl.ds(0, _RPC)], oz_h.at[c, pl.ds(r0, _RPC)])
        pltpu.sync_copy(accw.at[pl.ds(r0, _RPC)], wbuf.at[pl.ds(0, _RPC)])
        pltpu.sync_copy(wbuf.at[pl.ds(0, _RPC)], ow_h.at[c, pl.ds(r0, _RPC)])


_sc_conv = pl.kernel(
    _sc_body,
    out_type=[jax.ShapeDtypeStruct((_NC, _NPAD, _HD), jnp.float32),
              jax.ShapeDtypeStruct((_NC, _NPAD, _WW), jnp.float32)],
    mesh=plsc.VectorSubcoreMesh(core_axis_name="c", subcore_axis_name="s"),
    compiler_params=pltpu.CompilerParams(use_tc_tiling_on_sc=False,
                                         needs_layout_passes=False),
    scratch_types=(
        [pltpu.VMEM((_NCH, _CHUNK), jnp.int32),
         pltpu.VMEM((_NCH, _CHUNK), jnp.int32)]
        + [pltpu.VMEM((_CHUNK, _HD), jnp.float32),
           pltpu.VMEM((_CHUNK, 16), jnp.float32),
           pltpu.VMEM((_CHUNK, 16), jnp.float32)] * 2
        + [pltpu.VMEM((_CHUNK, _WW), jnp.float32),
           pltpu.VMEM_SHARED((_NPAD, _HD), jnp.float32),
           pltpu.VMEM_SHARED((_NPAD, _WW), jnp.float32),
           pltpu.SemaphoreType.DMA,
           pltpu.SemaphoreType.DMA]))


def _selectors():
    s1 = np.zeros((_HD, 16), np.float32)
    s2 = np.zeros((_HD, 16), np.float32)
    rs = np.zeros((16, _HD), np.float32)
    for h in range(_H):
        s1[h * _D:(h + 1) * _D, h] = 1.0
        s2[h * _D:(h + 1) * _D, _H + h] = 1.0
        rs[h, h * _D:(h + 1) * _D] = 1.0
    return jnp.asarray(s1), jnp.asarray(s2), jnp.asarray(rs)


def _prep_edges(ei):
    pad = jnp.zeros((_EPAD - _E,), jnp.int32)
    src = jnp.concatenate([ei[0], pad]).reshape(_NW, _NCH, _CHUNK)
    dst = jnp.concatenate([ei[1], pad]).reshape(_NW, _NCH, _CHUNK)
    return src, dst


def kernel(x, ei0, ei1, ei2,
           l0_W0, l0_al0, l0_ar0, l0_b0, l0_W1, l0_al1, l0_ar1, l0_b1,
           l0_W2, l0_al2, l0_ar2, l0_b2,
           l1_W0, l1_al0, l1_ar0, l1_b0, l1_W1, l1_al1, l1_ar1, l1_b1,
           l1_W2, l1_al2, l1_ar2, l1_b2, lin_W, lin_b):
    s1, s2, rsel = _selectors()
    edges = [_prep_edges(ei) for ei in (ei0, ei1, ei2)]

    def flat_params(ws, als, ars):
        out = []
        for w, al, ar in zip(ws, als, ars):
            out += [w, al.reshape(1, _HD), ar.reshape(1, _HD)]
        return out

    p0 = flat_params((l0_W0, l0_W1, l0_W2), (l0_al0, l0_al1, l0_al2),
                     (l0_ar0, l0_ar1, l0_ar2))
    p1 = flat_params((l1_W0, l1_W1, l1_W2), (l1_al0, l1_al1, l1_al2),
                     (l1_ar0, l1_ar1, l1_ar2))
    b0 = [b.reshape(1, _HD) for b in (l0_b0, l0_b1, l0_b2)]
    b1 = [b.reshape(1, _HD) for b in (l1_b0, l1_b1, l1_b2)]

    z0, e0, z1, e1, z2, e2 = _tc1(x, *p0, s1, s2)

    zaccs, waccs = [], []
    for (src, dst), z, e in zip(edges, (z0, z1, z2), (e0, e1, e2)):
        oz, ow = _sc_conv(z, e, src, dst)
        zaccs += [oz[0], oz[1]]
        waccs += [ow[0], ow[1]]

    z0, e0, z1, e1, z2, e2 = _tc2(*zaccs, *waccs, *b0, rsel, *p1, s1, s2)

    zaccs, waccs = [], []
    for (src, dst), z, e in zip(edges, (z0, z1, z2), (e0, e1, e2)):
        oz, ow = _sc_conv(z, e, src, dst)
        zaccs += [oz[0], oz[1]]
        waccs += [ow[0], ow[1]]

    lw = jnp.pad(lin_W, ((0, 0), (0, 256 - _C)))
    lb = jnp.pad(lin_b, (0, 256 - _C)).reshape(1, 256)
    out = _tc3(*zaccs, *waccs, *b1, rsel, lw, lb)
    return out[:, :_C]


# speedup vs baseline: 1.0079x; 1.0079x over previous
import jax
import jax.numpy as jnp
import numpy as np
from jax import lax
from jax.experimental import pallas as pl
from jax.experimental.pallas import tpu as pltpu
from jax.experimental.pallas import tpu_sc as plsc

_N = 10000
_E = 100000
_HD = 128
_H = 4
_D = 32
_C = 153
_NEG = 0.2
_EPS = 1e-9

_NC, _NS, _L = 2, 16, 16
_NW = _NC * _NS
_CHUNK = 96
_NCH = 34
_EPW = _NCH * _CHUNK
_EPAD = _NW * _EPW
_WW = 16
_NPAD = 10112
_RPT = _NPAD // _NS
_RPC = 79

_BLK = 400
_GRID = _N // _BLK


def _transform(hb, w, av, rv, s1, s2, zo, eo):
    z = jnp.dot(hb, w[...], preferred_element_type=jnp.float32)
    zo[...] = z
    eo[...] = (jnp.dot(z * av[...], s1[...], preferred_element_type=jnp.float32)
               + jnp.dot(z * rv[...], s2[...], preferred_element_type=jnp.float32))


def _tc1_body(h, w0, av0, rv0, w1, av1, rv1, w2, av2, rv2, s1, s2, zs, es):
    hb = h[...]
    for r, (w, av, rv) in enumerate(((w0, av0, rv0), (w1, av1, rv1),
                                     (w2, av2, rv2))):
        _transform(hb, w, av, rv, s1, s2, zs.at[r], es.at[r])


def _combine(za, wa, b0, b1, b2, rsel):
    out = b0[...] + b1[...] + b2[...]
    for r in range(3):
        nm = za[r, 0] + za[r, 1]
        den = jnp.dot(wa[r, 0] + wa[r, 1], rsel[...],
                      preferred_element_type=jnp.float32)
        out = out + nm / (den + _EPS)
    return out


def _tc2_body(za, wa, b0, b1, b2, rsel,
              w0, av0, rv0, w1, av1, rv1, w2, av2, rv2, s1, s2, zs, es):
    hb = jnp.maximum(_combine(za, wa, b0, b1, b2, rsel), 0.0)
    for r, (w, av, rv) in enumerate(((w0, av0, rv0), (w1, av1, rv1),
                                     (w2, av2, rv2))):
        _transform(hb, w, av, rv, s1, s2, zs.at[r], es.at[r])


def _tc3_body(za, wa, b0, b1, b2, rsel, lw, lb, out):
    hb = _combine(za, wa, b0, b1, b2, rsel)
    out[...] = jnp.dot(hb, lw[...], preferred_element_type=jnp.float32) + lb[...]


_FULL = lambda shape: pl.BlockSpec(shape, lambda i: (0,) * len(shape))
_ROWB = lambda shape: pl.BlockSpec(shape, lambda i: (i,) + (0,) * (len(shape) - 1))

_WSPECS = [_FULL((_HD, _HD)), _FULL((1, _HD)), _FULL((1, _HD))] * 3 + \
          [_FULL((_HD, 16)), _FULL((_HD, 16))]
_ZOUTS = [jax.ShapeDtypeStruct((3, _N, _HD), jnp.float32),
          jax.ShapeDtypeStruct((3, _N, 16), jnp.float32)]
_ZSPEC = [pl.BlockSpec((3, _BLK, _HD), lambda i: (0, i, 0)),
          pl.BlockSpec((3, _BLK, 16), lambda i: (0, i, 0))]
_ACCSPECS = [pl.BlockSpec((3, _NC, _BLK, _HD), lambda i: (0, 0, i, 0)),
             pl.BlockSpec((3, _NC, _BLK, _WW), lambda i: (0, 0, i, 0))] + \
            [_FULL((1, _HD))] * 3 + [_FULL((16, _HD))]

_tc1 = pl.pallas_call(
    _tc1_body, grid=(_GRID,),
    in_specs=[_ROWB((_BLK, _HD))] + _WSPECS,
    out_specs=_ZSPEC, out_shape=_ZOUTS)

_tc2 = pl.pallas_call(
    _tc2_body, grid=(_GRID,),
    in_specs=_ACCSPECS + _WSPECS,
    out_specs=_ZSPEC, out_shape=_ZOUTS)

_tc3 = pl.pallas_call(
    _tc3_body, grid=(_GRID,),
    in_specs=_ACCSPECS + [_FULL((_HD, 256)), _FULL((1, 256))],
    out_specs=_ROWB((_BLK, 256)),
    out_shape=jax.ShapeDtypeStruct((_N, 256), jnp.float32))


def _sc_body(z_h, eler_h, src_h, dst_h, oz_h, ow_h,
             sidx, didx, zbuf0, esb0, edb0, zbuf1, esb1, edb1,
             wbuf, accz, accw, sem0, sem1):
    c = lax.axis_index("c")
    s = lax.axis_index("s")
    wid = c * _NS + s
    zbuf = (zbuf0, zbuf1)
    esb = (esb0, esb1)
    edb = (edb0, edb1)
    sem = (sem0, sem1)

    for r in range(3):
        zr_h = z_h.at[r]
        er_h = eler_h.at[r]
        pltpu.sync_copy(src_h.at[r, wid], sidx)
        pltpu.sync_copy(dst_h.at[r, wid], didx)

        @plsc.parallel_loop(0, _CHUNK, unroll=4)
        def _zrow(i):
            for j in range(_HD // _L):
                zbuf0[i, pl.ds(j * _L, _L)] = jnp.zeros((_L,), jnp.float32)
            wbuf[i, pl.ds(0, _L)] = jnp.zeros((_L,), jnp.float32)
        for k in range(_RPT // _RPC):
            r0 = s * _RPT + k * _RPC
            pltpu.sync_copy(zbuf0.at[pl.ds(0, _RPC)], accz.at[pl.ds(r0, _RPC)])
            pltpu.sync_copy(wbuf.at[pl.ds(0, _RPC)], accw.at[pl.ds(r0, _RPC)])
        plsc.subcore_barrier()

        def _issue(j, b):
            pltpu.async_copy(zr_h.at[sidx.at[j]], zbuf[b], sem[b])
            pltpu.async_copy(er_h.at[sidx.at[j]], esb[b], sem[b])
            pltpu.async_copy(er_h.at[didx.at[j]], edb[b], sem[b])

        def _process(j, b):
            pltpu.make_async_copy(zr_h.at[sidx.at[j]], zbuf[b], sem[b]).wait()
            pltpu.make_async_copy(er_h.at[sidx.at[j]], esb[b], sem[b]).wait()
            pltpu.make_async_copy(er_h.at[didx.at[j]], edb[b], sem[b]).wait()
            gbase = wid * _EPW + j * _CHUNK
            for g in range(_CHUNK // _L):
                rows = lax.iota(jnp.int32, _L) + (g * _L)
                live = (gbase + g * _L + lax.iota(jnp.int32, _L)) < _E
                for h in range(_H):
                    a = plsc.load_gather(esb[b], [rows, jnp.full((_L,), h, jnp.int32)])
                    bb = plsc.load_gather(edb[b], [rows, jnp.full((_L,), _H + h, jnp.int32)])
                    e = a + bb
                    e = jnp.maximum(e, _NEG * e)
                    w = jnp.where(live, jnp.exp(e), 0.0)
                    plsc.store_scatter(wbuf, [rows, jnp.full((_L,), h, jnp.int32)], w)

            zb = zbuf[b]

            @plsc.parallel_loop(0, _CHUNK, unroll=4)
            def _scale(i):
                wrow = wbuf[i, pl.ds(0, _L)]
                for h in range(_H):
                    wv = wrow[h]
                    for tt in range(_D // _L):
                        sl = pl.ds(h * _D + tt * _L, _L)
                        zb[i, sl] = zb[i, sl] * wv

            pltpu.sync_copy(zbuf[b], accz.at[didx.at[j]], add=True)
            pltpu.sync_copy(wbuf, accw.at[didx.at[j]], add=True)

        _issue(0, 0)

        def _pair(jj, carry):
            j0 = jj * 2
            _issue(j0 + 1, 1)
            _process(j0, 0)

            @pl.when(jj < _NCH // 2 - 1)
            def _():
                _issue(j0 + 2, 0)
            _process(j0 + 1, 1)
            return carry
        lax.fori_loop(0, _NCH // 2, _pair, 0)

        plsc.subcore_barrier()
        for k in range(_RPT // _RPC):
            r0 = s * _RPT + k * _RPC
            pltpu.sync_copy(accz.at[pl.ds(r0, _RPC)], zbuf0.at[pl.ds(0, _RPC)])
            pltpu.sync_copy(zbuf0.at[pl.ds(0, _RPC)], oz_h.at[r, c, pl.ds(r0, _RPC)])
            pltpu.sync_copy(accw.at[pl.ds(r0, _RPC)], wbuf.at[pl.ds(0, _RPC)])
            pltpu.sync_copy(wbuf.at[pl.ds(0, _RPC)], ow_h.at[r, c, pl.ds(r0, _RPC)])


_sc_conv = pl.kernel(
    _sc_body,
    out_type=[jax.ShapeDtypeStruct((3, _NC, _NPAD, _HD), jnp.float32),
              jax.ShapeDtypeStruct((3, _NC, _NPAD, _WW), jnp.float32)],
    mesh=plsc.VectorSubcoreMesh(core_axis_name="c", subcore_axis_name="s"),
    compiler_params=pltpu.CompilerParams(use_tc_tiling_on_sc=False,
                                         needs_layout_passes=False),
    scratch_types=(
        [pltpu.VMEM((_NCH, _CHUNK), jnp.int32),
         pltpu.VMEM((_NCH, _CHUNK), jnp.int32)]
        + [pltpu.VMEM((_CHUNK, _HD), jnp.float32),
           pltpu.VMEM((_CHUNK, 16), jnp.float32),
           pltpu.VMEM((_CHUNK, 16), jnp.float32)] * 2
        + [pltpu.VMEM((_CHUNK, _WW), jnp.float32),
           pltpu.VMEM_SHARED((_NPAD, _HD), jnp.float32),
           pltpu.VMEM_SHARED((_NPAD, _WW), jnp.float32),
           pltpu.SemaphoreType.DMA,
           pltpu.SemaphoreType.DMA]))


def _selectors():
    s1 = np.zeros((_HD, 16), np.float32)
    s2 = np.zeros((_HD, 16), np.float32)
    rs = np.zeros((16, _HD), np.float32)
    for h in range(_H):
        s1[h * _D:(h + 1) * _D, h] = 1.0
        s2[h * _D:(h + 1) * _D, _H + h] = 1.0
        rs[h, h * _D:(h + 1) * _D] = 1.0
    return jnp.asarray(s1), jnp.asarray(s2), jnp.asarray(rs)


def _prep_edges(ei):
    pad = jnp.zeros((_EPAD - _E,), jnp.int32)
    src = jnp.concatenate([ei[0], pad]).reshape(_NW, _NCH, _CHUNK)
    dst = jnp.concatenate([ei[1], pad]).reshape(_NW, _NCH, _CHUNK)
    return src, dst


def kernel(x, ei0, ei1, ei2,
           l0_W0, l0_al0, l0_ar0, l0_b0, l0_W1, l0_al1, l0_ar1, l0_b1,
           l0_W2, l0_al2, l0_ar2, l0_b2,
           l1_W0, l1_al0, l1_ar0, l1_b0, l1_W1, l1_al1, l1_ar1, l1_b1,
           l1_W2, l1_al2, l1_ar2, l1_b2, lin_W, lin_b):
    s1, s2, rsel = _selectors()
    edges = [_prep_edges(ei) for ei in (ei0, ei1, ei2)]

    def flat_params(ws, als, ars):
        out = []
        for w, al, ar in zip(ws, als, ars):
            out += [w, al.reshape(1, _HD), ar.reshape(1, _HD)]
        return out

    p0 = flat_params((l0_W0, l0_W1, l0_W2), (l0_al0, l0_al1, l0_al2),
                     (l0_ar0, l0_ar1, l0_ar2))
    p1 = flat_params((l1_W0, l1_W1, l1_W2), (l1_al0, l1_al1, l1_al2),
                     (l1_ar0, l1_ar1, l1_ar2))
    b0 = [b.reshape(1, _HD) for b in (l0_b0, l0_b1, l0_b2)]
    b1 = [b.reshape(1, _HD) for b in (l1_b0, l1_b1, l1_b2)]

    srcs = jnp.stack([e[0] for e in edges])
    dsts = jnp.stack([e[1] for e in edges])

    zs, es = _tc1(x, *p0, s1, s2)
    oz, ow = _sc_conv(zs, es, srcs, dsts)
    zs, es = _tc2(oz, ow, *b0, rsel, *p1, s1, s2)
    oz, ow = _sc_conv(zs, es, srcs, dsts)

    lw = jnp.pad(lin_W, ((0, 0), (0, 256 - _C)))
    lb = jnp.pad(lin_b, (0, 256 - _C)).reshape(1, 256)
    out = _tc3(oz, ow, *b1, rsel, lw, lb)
    return out[:, :_C]
